# baseline (speedup 1.0000x reference)
import functools

import jax
import jax.numpy as jnp
from jax import lax
from jax.experimental import pallas as pl
from jax.experimental.pallas import tpu as pltpu
from jax.experimental.pallas import tpu_sc as plsc

N = 10000
NP = 10240
E = 320000
NC, NS = 2, 16
NW = NC * NS
EPW = 10240
EP = NW * EPW
K = 64
NCHUNK = EPW // K
KD = 64
NCHUNKD = EPW // KD
TRASH = N
RB = 512
GRID = NP // RB

@functools.cache
def _sc_mesh():
    return plsc.VectorSubcoreMesh(
        core_axis_name="c", subcore_axis_name="s", num_cores=NC, num_subcores=NS
    )


DW = 128


@functools.cache
def _make_deg_colp():
    return functools.partial(
        pl.kernel,
        out_type=(
            jax.ShapeDtypeStruct((NC, NP, DW), jnp.float32),
            jax.ShapeDtypeStruct((EP,), jnp.int32),
        ),
        mesh=_sc_mesh(),
        scratch_types=[
            pltpu.VMEM((EPW,), jnp.int32),
            pltpu.VMEM((EPW,), jnp.int32),
            pltpu.VMEM((EPW,), jnp.int32),
            pltpu.VMEM((KD,), jnp.int32),
            pltpu.VMEM((KD, DW), jnp.float32),
            pltpu.VMEM((16, DW), jnp.float32),
            pltpu.VMEM_SHARED((NP, DW), jnp.float32),
        ],
    )(_deg_colp_body)


def _deg_colp_body(row_hbm, col_hbm, deg_out, colp_out,
                   rowv, colv, colpv, cbuf, onesb, zb, acc):
    cid = lax.axis_index("c")
    sid = lax.axis_index("s")
    wid = sid * NC + cid
    base = wid * EPW
    pltpu.sync_copy(row_hbm.at[pl.ds(base, EPW)], rowv)
    pltpu.sync_copy(col_hbm.at[pl.ds(base, EPW)], colv)

    one16 = jnp.ones((16,), jnp.float32)
    z16 = jnp.zeros((16,), jnp.float32)

    for rr in range(KD):
        for kk in range(DW // 16):
            onesb[rr, pl.ds(kk * 16, 16)] = one16
    for rr in range(16):
        for kk in range(DW // 16):
            zb[rr, pl.ds(kk * 16, 16)] = z16

    rows_per_sub = NP // NS
    zbase = sid * rows_per_sub

    def zbody(i, c):
        pltpu.sync_copy(zb, acc.at[pl.ds(zbase + i * 16, 16)])
        return c

    lax.fori_loop(0, rows_per_sub // 16, zbody, 0)
    plsc.subcore_barrier()

    trash = jnp.full((16,), TRASH, jnp.int32)

    def body(j, c):
        for t in range(KD // 16):
            i = j * (KD // 16) + t
            r = rowv[pl.ds(i * 16, 16)]
            cc = colv[pl.ds(i * 16, 16)]
            cp = jnp.where(r != cc, cc, trash + (cc & 127))
            colpv[pl.ds(i * 16, 16)] = cp
            cbuf[pl.ds(t * 16, 16)] = cp
        pltpu.sync_copy(onesb, acc.at[cbuf], add=True)
        return c

    lax.fori_loop(0, NCHUNKD, body, 0)

    pltpu.sync_copy(colpv, colp_out.at[pl.ds(base, EPW)])
    plsc.subcore_barrier()
    pltpu.sync_copy(
        acc.at[pl.ds(zbase, rows_per_sub)],
        deg_out.at[cid, pl.ds(zbase, rows_per_sub)],
    )


NB = 4


@functools.cache
def _make_scatter(W, nin):
    @functools.partial(
        pl.kernel,
        out_type=tuple(
            jax.ShapeDtypeStruct((NC, NP, W), jnp.float32) for _ in range(nin)
        ),
        mesh=_sc_mesh(),
        scratch_types=(
            [pltpu.VMEM((K,), jnp.int32) for _ in range(NB)]
            + [pltpu.VMEM((K,), jnp.int32) for _ in range(NB)]
            + [pltpu.VMEM((K, W), jnp.float32) for _ in range(NB)]
            + [
                pltpu.VMEM((8, W), jnp.float32),
                pltpu.VMEM((K,), jnp.int32),
                pltpu.VMEM_SHARED((NP, W), jnp.float32),
            ]
            + [pltpu.SemaphoreType.DMA for _ in range(3 * NB)]
        ),
    )
    def _scatter(*args):
        yps = args[:nin]
        row_hbm, colp_hbm = args[nin], args[nin + 1]
        outs = args[nin + 2:2 * nin + 2]
        sc = list(args[2 * nin + 2:])
        rib = sc[0:NB]
        cib = sc[NB:2 * NB]
        gb = sc[2 * NB:3 * NB]
        zb = sc[3 * NB]
        dumv = sc[3 * NB + 1]
        acc = sc[3 * NB + 2]
        rsem = sc[3 * NB + 3:3 * NB + 3 + NB]
        csem = sc[3 * NB + 3 + NB:3 * NB + 3 + 2 * NB]
        gsem = sc[3 * NB + 3 + 2 * NB:3 * NB + 3 + 3 * NB]

        cid = lax.axis_index("c")
        sid = lax.axis_index("s")
        wid = sid * NC + cid
        base = wid * EPW

        z16 = jnp.zeros((16,), jnp.float32)
        for rr in range(8):
            for kk in range(W // 16):
                zb[rr, pl.ds(kk * 16, 16)] = z16

        rows_per_sub = NP // NS
        zbase = sid * rows_per_sub

        for h in range(nin):
            yp_hbm = yps[h]

            def zbody(i, c):
                pltpu.sync_copy(zb, acc.at[pl.ds(zbase + i * 8, 8)])
                return c

            lax.fori_loop(0, rows_per_sub // 8, zbody, 0)
            plsc.subcore_barrier()

            for b in range(NB):
                pltpu.async_copy(
                    row_hbm.at[pl.ds(base + b * K, K)], rib[b], rsem[b]
                )
                pltpu.async_copy(
                    colp_hbm.at[pl.ds(base + b * K, K)], cib[b], csem[b]
                )
            for b in range(NB):
                pltpu.make_async_copy(
                    row_hbm.at[pl.ds(0, K)], rib[b], rsem[b]
                ).wait()
                pltpu.async_copy(yp_hbm.at[rib[b]], gb[b], gsem[b])

            def mbody(jn, c):
                for b in range(NB):
                    pltpu.make_async_copy(
                        yp_hbm.at[dumv], gb[b], gsem[b]
                    ).wait()
                    pltpu.make_async_copy(
                        colp_hbm.at[pl.ds(0, K)], cib[b], csem[b]
                    ).wait()
                    pltpu.sync_copy(gb[b], acc.at[cib[b]], add=True)
                for b in range(NB):
                    j = jn * NB + b
                    pltpu.async_copy(
                        row_hbm.at[pl.ds(base + (j + NB) * K, K)],
                        rib[b], rsem[b],
                    )
                    pltpu.async_copy(
                        colp_hbm.at[pl.ds(base + (j + NB) * K, K)],
                        cib[b], csem[b],
                    )
                for b in range(NB):
                    pltpu.make_async_copy(
                        row_hbm.at[pl.ds(0, K)], rib[b], rsem[b]
                    ).wait()
                    pltpu.async_copy(yp_hbm.at[rib[b]], gb[b], gsem[b])
                return c

            lax.fori_loop(0, NCHUNK // NB - 1, mbody, 0)
            for b in range(NB):
                pltpu.make_async_copy(yp_hbm.at[dumv], gb[b], gsem[b]).wait()
                pltpu.make_async_copy(
                    colp_hbm.at[pl.ds(0, K)], cib[b], csem[b]
                ).wait()
                pltpu.sync_copy(gb[b], acc.at[cib[b]], add=True)
            plsc.subcore_barrier()
            pltpu.sync_copy(
                acc.at[pl.ds(zbase, rows_per_sub)],
                outs[h].at[cid, pl.ds(zbase, rows_per_sub)],
            )

    return _scatter


def _norm_body(degp_ref, dis_ref):
    p = degp_ref[...]
    deg = 1.0 + p[0, :, 0:1] + p[1, :, 0:1]
    dis_ref[...] = lax.rsqrt(deg)


def _mm_scale_body(x_ref, w_ref, dis_ref, y0_ref, y1_ref):
    dis = dis_ref[...]
    xw = jnp.dot(x_ref[...], w_ref[...], preferred_element_type=jnp.float32)
    y0_ref[...] = dis * xw[:, :128]
    y1_ref[...] = dis * xw[:, 128:]


def _mid_body(s0_ref, s1_ref, y0_ref, y1_ref, dis_ref,
              bi0_ref, ba0_ref, wi1_ref, wa1_ref, out_ref):
    dis = dis_ref[...]
    zi = dis * (s0_ref[0] + s0_ref[1] + y0_ref[...]) + bi0_ref[...]
    za = dis * (s1_ref[0] + s1_ref[1] + y1_ref[...]) + ba0_ref[...]
    hi = jnp.where(zi > 0, zi, jnp.exp(jnp.minimum(zi, 0.0)) - 1.0)
    ha = jnp.where(za > 0, za, jnp.exp(jnp.minimum(za, 0.0)) - 1.0)
    y2 = (jnp.dot(hi, wi1_ref[...], preferred_element_type=jnp.float32)
          + jnp.dot(ha, wa1_ref[...], preferred_element_type=jnp.float32))
    out_ref[...] = jnp.concatenate(
        [dis * y2, jnp.zeros_like(y2)], axis=1)


def _final_body(s2_ref, y2_ref, dis_ref, bi1_ref, ba1_ref, out_ref):
    dis = dis_ref[...]
    agg = s2_ref[0, :, 0:64] + s2_ref[1, :, 0:64] + y2_ref[:, 0:64]
    out_ref[...] = dis * agg + bi1_ref[...] + ba1_ref[...]


def kernel(x, edge_index, w_mul_p,
           W_i0, b_i0, Wp1_i0, Wp2_i0, bp2_i0,
           W_i1, b_i1, Wp1_i1, Wp2_i1, bp2_i1,
           W_a0, b_a0, Wp1_a0, Wp2_a0, bp2_a0,
           W_a1, b_a1, Wp1_a1, Wp2_a1, bp2_a1):
    row = edge_index[0]
    col = edge_index[1]
    padv = (jnp.arange(EP - E, dtype=jnp.int32) & 8191)
    rowp = jnp.concatenate([row, padv])
    colp_in = jnp.concatenate([col, padv])

    degp, colp = _make_deg_colp()(rowp, colp_in)

    dis_col = pl.pallas_call(
        _norm_body,
        grid=(GRID,),
        in_specs=[pl.BlockSpec((NC, RB, DW), lambda i: (0, i, 0))],
        out_specs=pl.BlockSpec((RB, 1), lambda i: (i, 0)),
        out_shape=jax.ShapeDtypeStruct((NP, 1), jnp.float32),
    )(degp)

    xp = jnp.zeros((NP, x.shape[1]), x.dtype).at[:N, :].set(x)
    wcat = jnp.concatenate([W_i0, W_a0], axis=0).T

    xwp0, xwp1 = pl.pallas_call(
        _mm_scale_body,
        grid=(GRID,),
        in_specs=[
            pl.BlockSpec((RB, 128), lambda i: (i, 0)),
            pl.BlockSpec((128, 256), lambda i: (0, 0)),
            pl.BlockSpec((RB, 1), lambda i: (i, 0)),
        ],
        out_specs=[
            pl.BlockSpec((RB, 128), lambda i: (i, 0)),
            pl.BlockSpec((RB, 128), lambda i: (i, 0)),
        ],
        out_shape=[
            jax.ShapeDtypeStruct((NP, 128), jnp.float32),
            jax.ShapeDtypeStruct((NP, 128), jnp.float32),
        ],
    )(xp, wcat, dis_col)

    s_h0, s_h1 = _make_scatter(128, 2)(xwp0, xwp1, rowp, colp)

    y2p = pl.pallas_call(
        _mid_body,
        grid=(GRID,),
        in_specs=[
            pl.BlockSpec((NC, RB, 128), lambda i: (0, i, 0)),
            pl.BlockSpec((NC, RB, 128), lambda i: (0, i, 0)),
            pl.BlockSpec((RB, 128), lambda i: (i, 0)),
            pl.BlockSpec((RB, 128), lambda i: (i, 0)),
            pl.BlockSpec((RB, 1), lambda i: (i, 0)),
            pl.BlockSpec((1, 128), lambda i: (0, 0)),
            pl.BlockSpec((1, 128), lambda i: (0, 0)),
            pl.BlockSpec((128, 64), lambda i: (0, 0)),
            pl.BlockSpec((128, 64), lambda i: (0, 0)),
        ],
        out_specs=pl.BlockSpec((RB, 128), lambda i: (i, 0)),
        out_shape=jax.ShapeDtypeStruct((NP, 128), jnp.float32),
    )(s_h0, s_h1, xwp0, xwp1, dis_col,
      b_i0.reshape(1, 128), b_a0.reshape(1, 128), W_i1.T, W_a1.T)

    (s2,) = _make_scatter(128, 1)(y2p, rowp, colp)

    FB = 400
    out = pl.pallas_call(
        _final_body,
        grid=(N // FB,),
        in_specs=[
            pl.BlockSpec((NC, FB, 128), lambda i: (0, i, 0)),
            pl.BlockSpec((FB, 128), lambda i: (i, 0)),
            pl.BlockSpec((FB, 1), lambda i: (i, 0)),
            pl.BlockSpec((1, 64), lambda i: (0, 0)),
            pl.BlockSpec((1, 64), lambda i: (0, 0)),
        ],
        out_specs=pl.BlockSpec((FB, 64), lambda i: (i, 0)),
        out_shape=jax.ShapeDtypeStruct((N, 64), jnp.float32),
    )(s2, y2p, dis_col, b_i1.reshape(1, 64), b_a1.reshape(1, 64))

    return out


# speedup vs baseline: 1.2424x; 1.2424x over previous
import functools

import jax
import jax.numpy as jnp
from jax import lax
from jax.experimental import pallas as pl
from jax.experimental.pallas import tpu as pltpu
from jax.experimental.pallas import tpu_sc as plsc

N = 10000
NP = 10240
E = 320000
NC, NS = 2, 16
NW = NC * NS
EPW = 10240
EP = NW * EPW
K = 80
NCHUNK = EPW // K
KD = 64
NCHUNKD = EPW // KD
TRASH = N
RB = 512
GRID = NP // RB

@functools.cache
def _sc_mesh():
    return plsc.VectorSubcoreMesh(
        core_axis_name="c", subcore_axis_name="s", num_cores=NC, num_subcores=NS
    )


DW = 128


@functools.cache
def _make_deg_colp():
    return functools.partial(
        pl.kernel,
        out_type=(
            jax.ShapeDtypeStruct((NC, NP, DW), jnp.float32),
            jax.ShapeDtypeStruct((EP,), jnp.int32),
        ),
        mesh=_sc_mesh(),
        scratch_types=[
            pltpu.VMEM((EPW,), jnp.int32),
            pltpu.VMEM((EPW,), jnp.int32),
            pltpu.VMEM((EPW,), jnp.int32),
            pltpu.VMEM((KD,), jnp.int32),
            pltpu.VMEM((KD, DW), jnp.float32),
            pltpu.VMEM((16, DW), jnp.float32),
            pltpu.VMEM_SHARED((NP, DW), jnp.float32),
        ],
    )(_deg_colp_body)


def _deg_colp_body(row_hbm, col_hbm, deg_out, colp_out,
                   rowv, colv, colpv, cbuf, onesb, zb, acc):
    cid = lax.axis_index("c")
    sid = lax.axis_index("s")
    wid = sid * NC + cid
    base = wid * EPW
    pltpu.sync_copy(row_hbm.at[pl.ds(base, EPW)], rowv)
    pltpu.sync_copy(col_hbm.at[pl.ds(base, EPW)], colv)

    one16 = jnp.ones((16,), jnp.float32)
    z16 = jnp.zeros((16,), jnp.float32)

    for rr in range(KD):
        for kk in range(DW // 16):
            onesb[rr, pl.ds(kk * 16, 16)] = one16
    for rr in range(16):
        for kk in range(DW // 16):
            zb[rr, pl.ds(kk * 16, 16)] = z16

    rows_per_sub = NP // NS
    zbase = sid * rows_per_sub

    def zbody(i, c):
        pltpu.sync_copy(zb, acc.at[pl.ds(zbase + i * 16, 16)])
        return c

    lax.fori_loop(0, rows_per_sub // 16, zbody, 0)
    plsc.subcore_barrier()

    trash = jnp.full((16,), TRASH, jnp.int32)

    def body(j, c):
        for t in range(KD // 16):
            i = j * (KD // 16) + t
            r = rowv[pl.ds(i * 16, 16)]
            cc = colv[pl.ds(i * 16, 16)]
            cp = jnp.where(r != cc, cc, trash + (cc & 127))
            colpv[pl.ds(i * 16, 16)] = cp
            cbuf[pl.ds(t * 16, 16)] = cp
        pltpu.sync_copy(onesb, acc.at[cbuf], add=True)
        return c

    lax.fori_loop(0, NCHUNKD, body, 0)

    pltpu.sync_copy(colpv, colp_out.at[pl.ds(base, EPW)])
    plsc.subcore_barrier()
    pltpu.sync_copy(
        acc.at[pl.ds(zbase, rows_per_sub)],
        deg_out.at[cid, pl.ds(zbase, rows_per_sub)],
    )


NB = 2


@functools.cache
def _make_scatter(W, nin):
    @functools.partial(
        pl.kernel,
        out_type=tuple(
            jax.ShapeDtypeStruct((NC, NP, W), jnp.float32) for _ in range(nin)
        ),
        mesh=_sc_mesh(),
        scratch_types=(
            [
                pltpu.VMEM((EPW,), jnp.int32),
                pltpu.VMEM((NCHUNK, K), jnp.int32),
            ]
            + [pltpu.VMEM((K, W), jnp.float32) for _ in range(NB)]
            + [
                pltpu.VMEM((8, W), jnp.float32),
                pltpu.VMEM((K,), jnp.int32),
                pltpu.VMEM_SHARED((NP, W), jnp.float32),
            ]
            + [pltpu.SemaphoreType.DMA for _ in range(NB)]
        ),
    )
    def _scatter(*args):
        yps = args[:nin]
        row_hbm, colp_hbm = args[nin], args[nin + 1]
        outs = args[nin + 2:2 * nin + 2]
        sc = list(args[2 * nin + 2:])
        rowv, colpv = sc[0], sc[1]
        gb = sc[2:2 + NB]
        zb = sc[2 + NB]
        dumv = sc[3 + NB]
        acc = sc[4 + NB]
        gsem = sc[5 + NB:5 + 2 * NB]

        cid = lax.axis_index("c")
        sid = lax.axis_index("s")
        wid = sid * NC + cid
        base = wid * EPW
        pltpu.sync_copy(row_hbm.at[pl.ds(base, EPW)], rowv)
        pltpu.sync_copy(colp_hbm.at[wid], colpv)

        z16 = jnp.zeros((16,), jnp.float32)
        for rr in range(8):
            for kk in range(W // 16):
                zb[rr, pl.ds(kk * 16, 16)] = z16

        rows_per_sub = NP // NS
        zbase = sid * rows_per_sub

        for h in range(nin):
            yp_hbm = yps[h]

            def zbody(i, c):
                pltpu.sync_copy(zb, acc.at[pl.ds(zbase + i * 8, 8)])
                return c

            lax.fori_loop(0, rows_per_sub // 8, zbody, 0)
            plsc.subcore_barrier()

            for b in range(NB):
                pltpu.async_copy(
                    yp_hbm.at[rowv.at[pl.ds(b * K, K)]], gb[b], gsem[b]
                )

            def mbody(jn, c):
                for b in range(NB):
                    j = jn * NB + b
                    pltpu.make_async_copy(
                        yp_hbm.at[dumv], gb[b], gsem[b]
                    ).wait()
                    pltpu.sync_copy(gb[b], acc.at[colpv.at[j]], add=True)

                    @pl.when(j + NB < NCHUNK)
                    def _():
                        pltpu.async_copy(
                            yp_hbm.at[rowv.at[pl.ds((j + NB) * K, K)]],
                            gb[b], gsem[b],
                        )

                return c

            lax.fori_loop(0, NCHUNK // NB, mbody, 0)
            plsc.subcore_barrier()
            pltpu.sync_copy(
                acc.at[pl.ds(zbase, rows_per_sub)],
                outs[h].at[cid, pl.ds(zbase, rows_per_sub)],
            )

    return _scatter


def _norm_body(degp_ref, dis_ref):
    p = degp_ref[...]
    deg = 1.0 + p[0, :, 0:1] + p[1, :, 0:1]
    dis_ref[...] = lax.rsqrt(deg)


def _mm_scale_body(x_ref, w_ref, dis_ref, y0_ref, y1_ref):
    dis = dis_ref[...]
    xw = jnp.dot(x_ref[...], w_ref[...], preferred_element_type=jnp.float32)
    y0_ref[...] = dis * xw[:, :128]
    y1_ref[...] = dis * xw[:, 128:]


def _mid_body(s0_ref, s1_ref, y0_ref, y1_ref, dis_ref,
              bi0_ref, ba0_ref, wi1_ref, wa1_ref, out_ref):
    dis = dis_ref[...]
    zi = dis * (s0_ref[0] + s0_ref[1] + y0_ref[...]) + bi0_ref[...]
    za = dis * (s1_ref[0] + s1_ref[1] + y1_ref[...]) + ba0_ref[...]
    hi = jnp.where(zi > 0, zi, jnp.exp(jnp.minimum(zi, 0.0)) - 1.0)
    ha = jnp.where(za > 0, za, jnp.exp(jnp.minimum(za, 0.0)) - 1.0)
    y2 = (jnp.dot(hi, wi1_ref[...], preferred_element_type=jnp.float32)
          + jnp.dot(ha, wa1_ref[...], preferred_element_type=jnp.float32))
    out_ref[...] = jnp.concatenate(
        [dis * y2, jnp.zeros_like(y2)], axis=1)


def _final_body(s2_ref, y2_ref, dis_ref, bi1_ref, ba1_ref, out_ref):
    dis = dis_ref[...]
    agg = s2_ref[0, :, 0:64] + s2_ref[1, :, 0:64] + y2_ref[:, 0:64]
    out_ref[...] = dis * agg + bi1_ref[...] + ba1_ref[...]


def kernel(x, edge_index, w_mul_p,
           W_i0, b_i0, Wp1_i0, Wp2_i0, bp2_i0,
           W_i1, b_i1, Wp1_i1, Wp2_i1, bp2_i1,
           W_a0, b_a0, Wp1_a0, Wp2_a0, bp2_a0,
           W_a1, b_a1, Wp1_a1, Wp2_a1, bp2_a1):
    row = edge_index[0]
    col = edge_index[1]
    padv = (jnp.arange(EP - E, dtype=jnp.int32) & 8191)
    rowp = jnp.concatenate([row, padv])
    colp_in = jnp.concatenate([col, padv])

    degp, colp = _make_deg_colp()(rowp, colp_in)
    colp2 = colp.reshape(NW, NCHUNK, K)

    dis_col = pl.pallas_call(
        _norm_body,
        grid=(GRID,),
        in_specs=[pl.BlockSpec((NC, RB, DW), lambda i: (0, i, 0))],
        out_specs=pl.BlockSpec((RB, 1), lambda i: (i, 0)),
        out_shape=jax.ShapeDtypeStruct((NP, 1), jnp.float32),
    )(degp)

    xp = jnp.zeros((NP, x.shape[1]), x.dtype).at[:N, :].set(x)
    wcat = jnp.concatenate([W_i0, W_a0], axis=0).T

    xwp0, xwp1 = pl.pallas_call(
        _mm_scale_body,
        grid=(GRID,),
        in_specs=[
            pl.BlockSpec((RB, 128), lambda i: (i, 0)),
            pl.BlockSpec((128, 256), lambda i: (0, 0)),
            pl.BlockSpec((RB, 1), lambda i: (i, 0)),
        ],
        out_specs=[
            pl.BlockSpec((RB, 128), lambda i: (i, 0)),
            pl.BlockSpec((RB, 128), lambda i: (i, 0)),
        ],
        out_shape=[
            jax.ShapeDtypeStruct((NP, 128), jnp.float32),
            jax.ShapeDtypeStruct((NP, 128), jnp.float32),
        ],
    )(xp, wcat, dis_col)

    s_h0, s_h1 = _make_scatter(128, 2)(xwp0, xwp1, rowp, colp2)

    y2p = pl.pallas_call(
        _mid_body,
        grid=(GRID,),
        in_specs=[
            pl.BlockSpec((NC, RB, 128), lambda i: (0, i, 0)),
            pl.BlockSpec((NC, RB, 128), lambda i: (0, i, 0)),
            pl.BlockSpec((RB, 128), lambda i: (i, 0)),
            pl.BlockSpec((RB, 128), lambda i: (i, 0)),
            pl.BlockSpec((RB, 1), lambda i: (i, 0)),
            pl.BlockSpec((1, 128), lambda i: (0, 0)),
            pl.BlockSpec((1, 128), lambda i: (0, 0)),
            pl.BlockSpec((128, 64), lambda i: (0, 0)),
            pl.BlockSpec((128, 64), lambda i: (0, 0)),
        ],
        out_specs=pl.BlockSpec((RB, 128), lambda i: (i, 0)),
        out_shape=jax.ShapeDtypeStruct((NP, 128), jnp.float32),
    )(s_h0, s_h1, xwp0, xwp1, dis_col,
      b_i0.reshape(1, 128), b_a0.reshape(1, 128), W_i1.T, W_a1.T)

    (s2,) = _make_scatter(128, 1)(y2p, rowp, colp2)

    FB = 400
    out = pl.pallas_call(
        _final_body,
        grid=(N // FB,),
        in_specs=[
            pl.BlockSpec((NC, FB, 128), lambda i: (0, i, 0)),
            pl.BlockSpec((FB, 128), lambda i: (i, 0)),
            pl.BlockSpec((FB, 1), lambda i: (i, 0)),
            pl.BlockSpec((1, 64), lambda i: (0, 0)),
            pl.BlockSpec((1, 64), lambda i: (0, 0)),
        ],
        out_specs=pl.BlockSpec((FB, 64), lambda i: (i, 0)),
        out_shape=jax.ShapeDtypeStruct((N, 64), jnp.float32),
    )(s2, y2p, dis_col, b_i1.reshape(1, 64), b_a1.reshape(1, 64))

    return out
